# baseline (speedup 1.0000x reference)
import functools

import jax
import jax.numpy as jnp
from jax import lax
from jax.experimental import pallas as pl
from jax.experimental.pallas import tpu as pltpu
from jax.experimental.pallas import tpu_sc as plsc

_N = 10000
_E = 320000
_H = 128
_B = 16
_NW = 32
_EPW = _E // _NW
_CH = 80
_NCH = _EPW // _CH
_NP = 10240
_RPT = _NP // 16

_SC_CACHE = {}


def _sc_kernels():
    if "deg" in _SC_CACHE:
        return _SC_CACHE["deg"], _SC_CACHE["msg"]

    mesh = plsc.VectorSubcoreMesh(core_axis_name="c", subcore_axis_name="s")

    @functools.partial(
        pl.kernel,
        mesh=mesh,
        out_type=jax.ShapeDtypeStruct((2, 2, _NP, _H), jnp.float32),
        scratch_types=[
            pltpu.VMEM((_CH,), jnp.int32),
            pltpu.VMEM((_CH, _H), jnp.float32),
            pltpu.VMEM_SHARED((_NP, _H), jnp.float32),
        ],
    )
    def deg_kernel(src_hbm, dst_hbm, ones_hbm, zero_hbm, out_hbm,
                   idxv, ones_v, acc):
        cid = lax.axis_index("c")
        sid = lax.axis_index("s")
        pltpu.sync_copy(ones_hbm, ones_v)
        base = (cid * 16 + sid) * _EPW
        my_rows = pl.ds(sid * _RPT, _RPT)

        for which, e_hbm in ((0, src_hbm), (1, dst_hbm)):
            pltpu.sync_copy(zero_hbm, acc.at[my_rows])
            plsc.subcore_barrier()

            def body(g, carry):
                pltpu.sync_copy(e_hbm.at[pl.ds(base + g * _CH, _CH)], idxv)
                pltpu.sync_copy(ones_v, acc.at[idxv], add=True)
                return carry

            lax.fori_loop(0, _NCH, body, 0)
            plsc.subcore_barrier()
            pltpu.sync_copy(acc.at[my_rows],
                            out_hbm.at[cid, which, my_rows])

    @functools.partial(
        pl.kernel,
        mesh=mesh,
        out_type=jax.ShapeDtypeStruct((2, _NP, _H), jnp.float32),
        scratch_types=[
            pltpu.VMEM((_EPW,), jnp.int32),
            pltpu.VMEM((_NCH, _CH), jnp.int32),
            pltpu.VMEM((_CH, _H), jnp.float32),
            pltpu.VMEM((_CH, _H), jnp.float32),
            pltpu.VMEM_SHARED((_NP, _H), jnp.float32),
            pltpu.SemaphoreType.DMA,
            pltpu.SemaphoreType.DMA,
            pltpu.SemaphoreType.DMA,
            pltpu.SemaphoreType.DMA,
        ],
    )
    def msg_kernel(h_hbm, src_hbm, dst_hbm, zero_hbm, out_hbm, srcv, dstm,
                   rows0, rows1, acc, sem0, sem1, ssem0, ssem1):
        cid = lax.axis_index("c")
        sid = lax.axis_index("s")
        wid = cid * 16 + sid
        pltpu.sync_copy(src_hbm.at[pl.ds(wid * _EPW, _EPW)], srcv)
        pltpu.sync_copy(dst_hbm.at[wid], dstm)
        pltpu.sync_copy(zero_hbm, acc.at[pl.ds(sid * _RPT, _RPT)])
        plsc.subcore_barrier()

        def gather(g, rows, sem):
            return pltpu.async_copy(h_hbm.at[srcv.at[pl.ds(g * _CH, _CH)]],
                                    rows, sem)

        def gwait(g, rows, sem):
            pltpu.make_async_copy(h_hbm.at[srcv.at[pl.ds(g * _CH, _CH)]],
                                  rows, sem).wait()

        def scat(g, rows, sem):
            pltpu.async_copy(rows, acc.at[dstm.at[g]], sem, add=True)

        def swait(g, rows, sem):
            pltpu.make_async_copy(rows, acc.at[dstm.at[g]], sem).wait()

        gather(0, rows0, sem0)
        gather(1, rows1, sem1)

        def pair(t, carry):
            g0 = 2 * t
            g1 = g0 + 1
            gwait(g0, rows0, sem0)
            scat(g0, rows0, ssem0)
            gwait(g1, rows1, sem1)
            scat(g1, rows1, ssem1)
            swait(g0, rows0, ssem0)
            gather(g0 + 2, rows0, sem0)
            swait(g1, rows1, ssem1)
            gather(g1 + 2, rows1, sem1)
            return carry

        lax.fori_loop(0, (_NCH - 3) // 2, pair, 0)
        gwait(_NCH - 3, rows0, sem0)
        scat(_NCH - 3, rows0, ssem0)
        gwait(_NCH - 2, rows1, sem1)
        scat(_NCH - 2, rows1, ssem1)
        swait(_NCH - 3, rows0, ssem0)
        gather(_NCH - 1, rows0, sem0)
        gwait(_NCH - 1, rows0, sem0)
        scat(_NCH - 1, rows0, ssem0)
        swait(_NCH - 1, rows0, ssem0)
        swait(_NCH - 2, rows1, ssem1)

        plsc.subcore_barrier()
        pltpu.sync_copy(acc.at[pl.ds(sid * _RPT, _RPT)],
                        out_hbm.at[cid, pl.ds(sid * _RPT, _RPT)])

    _SC_CACHE["deg"] = deg_kernel
    _SC_CACHE["msg"] = msg_kernel
    return deg_kernel, msg_kernel


def _round0_body(x_ref, W_ref, dgo_ref, dgi_ref, h_ref, dO_ref, dI_ref):
    d_out = jnp.sum(dgo_ref[...], axis=0)
    d_in = jnp.sum(dgi_ref[...], axis=0)
    dO = lax.rsqrt(jnp.maximum(d_out, 1.0)).reshape(_N, 1)
    dI = lax.rsqrt(jnp.maximum(d_in, 1.0)).reshape(_N, 1)
    dO_ref[...] = dO
    dI_ref[...] = dI
    h_ref[...] = jnp.dot(x_ref[...] * dO, W_ref[...],
                         preferred_element_type=jnp.float32)


def _round_body(relu, parts_ref, dI_ref, dO_ref, b_ref, W_ref, h_ref):
    f = parts_ref[0, :_N, :] + parts_ref[1, :_N, :]
    f = f * dI_ref[...] + b_ref[...]
    if relu:
        f = jnp.maximum(f, 0.0)
    h_ref[...] = jnp.dot(f * dO_ref[...], W_ref[...],
                         preferred_element_type=jnp.float32)


def _epilogue_body(parts_ref, dI_ref, b_ref, gid_ref, add_ref,
                   w1a_ref, w1b_ref, b1_ref, w2_ref, b2_ref, w3_ref, b3_ref,
                   out_ref):
    f = parts_ref[0, :_N, :] + parts_ref[1, :_N, :]
    f = jnp.maximum(f * dI_ref[...] + b_ref[...], 0.0)
    iota = lax.broadcasted_iota(jnp.int32, (_B, 1), 0)
    onehot = (gid_ref[...] == iota).astype(jnp.float32)
    sums = jnp.dot(onehot, f, preferred_element_type=jnp.float32)
    counts = jnp.maximum(jnp.sum(onehot, axis=1), 1.0).reshape(_B, 1)
    mean = sums / counts
    h1 = (jnp.dot(mean, w1a_ref[...], preferred_element_type=jnp.float32)
          + jnp.dot(add_ref[...], w1b_ref[...], preferred_element_type=jnp.float32)
          + b1_ref[...])
    h1 = jnp.where(h1 > 0, h1, 0.01 * h1)
    h2 = jnp.dot(h1, w2_ref[...], preferred_element_type=jnp.float32) + b2_ref[...]
    h2 = jnp.where(h2 > 0, h2, 0.01 * h2)
    out_ref[...] = jnp.dot(h2, w3_ref[...],
                           preferred_element_type=jnp.float32) + b3_ref[...]


_f32 = jnp.float32

_round0_call = pl.pallas_call(
    _round0_body,
    out_shape=[jax.ShapeDtypeStruct((_N, _H), _f32),
               jax.ShapeDtypeStruct((_N, 1), _f32),
               jax.ShapeDtypeStruct((_N, 1), _f32)],
)
_round_relu_call = pl.pallas_call(
    functools.partial(_round_body, True),
    out_shape=jax.ShapeDtypeStruct((_N, _H), _f32),
)
_round_norelu_call = pl.pallas_call(
    functools.partial(_round_body, False),
    out_shape=jax.ShapeDtypeStruct((_N, _H), _f32),
)
_epilogue_call = pl.pallas_call(
    _epilogue_body,
    out_shape=jax.ShapeDtypeStruct((_B, 1), _f32),
)


def kernel(x, edge_index, graph_ids, add_features, W0, b0, gcr_W, gcr_b,
           rW1, rb1, rW2, rb2, rW3, rb3):
    src = edge_index[0]
    dst = edge_index[1]
    _deg_kernel, _msg_kernel = _sc_kernels()

    ones_rows = jnp.ones((_CH, _H), _f32)
    zeros = jnp.zeros((_RPT, _H), _f32)
    degp = _deg_kernel(src, dst, ones_rows, zeros)
    dgo = degp[:, 0, :_N, 0]
    dgi = degp[:, 1, :_N, 0]

    h, dO, dI = _round0_call(x, W0, dgo, dgi)
    dst3 = dst.reshape(_NW, _NCH, _CH)

    parts = _msg_kernel(h, src, dst3, zeros)
    for k in range(1, 11):
        i, j = divmod(k - 1, 2)
        W = gcr_W[i, j]
        b_prev = b0.reshape(1, _H) if k == 1 else gcr_b[(k - 2) // 2, (k - 2) % 2].reshape(1, _H)
        call = _round_norelu_call if k == 1 else _round_relu_call
        h = call(parts, dI, dO, b_prev, W)
        parts = _msg_kernel(h, src, dst3, zeros)

    b_last = gcr_b[4, 1].reshape(1, _H)
    gid_row = graph_ids.reshape(1, _N)
    out = _epilogue_call(parts, dI, b_last, gid_row, add_features,
                         rW1[:_H], rW1[_H:], rb1.reshape(1, 1024),
                         rW2, rb2.reshape(1, 512), rW3, rb3.reshape(1, 1))
    return out[:, 0]


# speedup vs baseline: 1.2854x; 1.2854x over previous
import functools

import jax
import jax.numpy as jnp
from jax import lax
from jax.experimental import pallas as pl
from jax.experimental.pallas import tpu as pltpu
from jax.experimental.pallas import tpu_sc as plsc

_N = 10000
_E = 320000
_H = 128
_B = 16
_NW = 32
_EPW = _E // _NW
_CH = 80
_NCH = _EPW // _CH
_NP = 10240
_RPT = _NP // 16

_SC_CACHE = {}


def _sc_kernels():
    if "deg" in _SC_CACHE:
        return _SC_CACHE["deg"], _SC_CACHE["msg"]

    mesh = plsc.VectorSubcoreMesh(core_axis_name="c", subcore_axis_name="s")

    @functools.partial(
        pl.kernel,
        mesh=mesh,
        out_type=jax.ShapeDtypeStruct((2, 2, _NP, _H), jnp.float32),
        scratch_types=[
            pltpu.VMEM((_NCH, _CH), jnp.int32),
            pltpu.VMEM((_NCH, _CH), jnp.int32),
            pltpu.VMEM((_CH, _H), jnp.float32),
            pltpu.VMEM_SHARED((_NP, _H), jnp.float32),
            pltpu.SemaphoreType.DMA,
        ],
    )
    def deg_kernel(src_hbm, dst_hbm, ones_hbm, zero_hbm, out_hbm,
                   srcm, dstm, ones_v, acc, sem):
        cid = lax.axis_index("c")
        sid = lax.axis_index("s")
        wid = cid * 16 + sid
        pltpu.sync_copy(ones_hbm, ones_v)
        pltpu.sync_copy(src_hbm.at[wid], srcm)
        pltpu.sync_copy(dst_hbm.at[wid], dstm)
        my_rows = pl.ds(sid * _RPT, _RPT)

        for which, em in ((0, srcm), (1, dstm)):
            pltpu.sync_copy(zero_hbm, acc.at[my_rows])
            plsc.subcore_barrier()

            def scat(g):
                pltpu.async_copy(ones_v, acc.at[em.at[g]], sem, add=True)

            def swait(g):
                pltpu.make_async_copy(ones_v, acc.at[em.at[g]], sem).wait()

            for g in range(4):
                scat(g)

            def body(g, carry):
                scat(g + 4)
                swait(g)
                return carry

            lax.fori_loop(0, _NCH - 4, body, 0)
            for g in range(_NCH - 4, _NCH):
                swait(g)
            plsc.subcore_barrier()
            pltpu.sync_copy(acc.at[my_rows],
                            out_hbm.at[cid, which, my_rows])

    @functools.partial(
        pl.kernel,
        mesh=mesh,
        out_type=jax.ShapeDtypeStruct((2, _NP, _H), jnp.float32),
        scratch_types=[
            pltpu.VMEM((_EPW,), jnp.int32),
            pltpu.VMEM((_NCH, _CH), jnp.int32),
            pltpu.VMEM((_CH, _H), jnp.float32),
            pltpu.VMEM((_CH, _H), jnp.float32),
            pltpu.VMEM_SHARED((_NP, _H), jnp.float32),
            pltpu.SemaphoreType.DMA,
            pltpu.SemaphoreType.DMA,
            pltpu.SemaphoreType.DMA,
            pltpu.SemaphoreType.DMA,
        ],
    )
    def msg_kernel(h_hbm, src_hbm, dst_hbm, zero_hbm, out_hbm, srcv, dstm,
                   rows0, rows1, acc, sem0, sem1, ssem0, ssem1):
        cid = lax.axis_index("c")
        sid = lax.axis_index("s")
        wid = cid * 16 + sid
        pltpu.sync_copy(src_hbm.at[pl.ds(wid * _EPW, _EPW)], srcv)
        pltpu.sync_copy(dst_hbm.at[wid], dstm)
        pltpu.sync_copy(zero_hbm, acc.at[pl.ds(sid * _RPT, _RPT)])
        plsc.subcore_barrier()

        def gather(g, rows, sem):
            return pltpu.async_copy(h_hbm.at[srcv.at[pl.ds(g * _CH, _CH)]],
                                    rows, sem)

        def gwait(g, rows, sem):
            pltpu.make_async_copy(h_hbm.at[srcv.at[pl.ds(g * _CH, _CH)]],
                                  rows, sem).wait()

        def scat(g, rows):
            pltpu.sync_copy(rows, acc.at[dstm.at[g]], add=True)

        gather(0, rows0, sem0)

        def pair(t, carry):
            g0 = 2 * t
            gather(g0 + 1, rows1, sem1)
            gwait(g0, rows0, sem0)
            scat(g0, rows0)
            gather(g0 + 2, rows0, sem0)
            gwait(g0 + 1, rows1, sem1)
            scat(g0 + 1, rows1)
            return carry

        lax.fori_loop(0, (_NCH - 1) // 2, pair, 0)
        gwait(_NCH - 1, rows0, sem0)
        scat(_NCH - 1, rows0)

        plsc.subcore_barrier()
        pltpu.sync_copy(acc.at[pl.ds(sid * _RPT, _RPT)],
                        out_hbm.at[cid, pl.ds(sid * _RPT, _RPT)])

    _SC_CACHE["deg"] = deg_kernel
    _SC_CACHE["msg"] = msg_kernel
    return deg_kernel, msg_kernel


def _round0_body(x_ref, W_ref, dgo_ref, dgi_ref, h_ref, dO_ref, dI_ref):
    d_out = jnp.sum(dgo_ref[...], axis=0)
    d_in = jnp.sum(dgi_ref[...], axis=0)
    dO = lax.rsqrt(jnp.maximum(d_out, 1.0)).reshape(_N, 1)
    dI = lax.rsqrt(jnp.maximum(d_in, 1.0)).reshape(_N, 1)
    dO_ref[...] = dO
    dI_ref[...] = dI
    h_ref[...] = jnp.dot(x_ref[...] * dO, W_ref[...],
                         preferred_element_type=jnp.float32)


def _round_body(relu, parts_ref, dI_ref, dO_ref, b_ref, W_ref, h_ref):
    f = parts_ref[0, :_N, :] + parts_ref[1, :_N, :]
    f = f * dI_ref[...] + b_ref[...]
    if relu:
        f = jnp.maximum(f, 0.0)
    h_ref[...] = jnp.dot(f * dO_ref[...], W_ref[...],
                         preferred_element_type=jnp.float32)


def _epilogue_body(parts_ref, dI_ref, b_ref, gid_ref, add_ref,
                   w1a_ref, w1b_ref, b1_ref, w2_ref, b2_ref, w3_ref, b3_ref,
                   out_ref):
    f = parts_ref[0, :_N, :] + parts_ref[1, :_N, :]
    f = jnp.maximum(f * dI_ref[...] + b_ref[...], 0.0)
    iota = lax.broadcasted_iota(jnp.int32, (_B, 1), 0)
    onehot = (gid_ref[...] == iota).astype(jnp.float32)
    sums = jnp.dot(onehot, f, preferred_element_type=jnp.float32)
    counts = jnp.maximum(jnp.sum(onehot, axis=1), 1.0).reshape(_B, 1)
    mean = sums / counts
    h1 = (jnp.dot(mean, w1a_ref[...], preferred_element_type=jnp.float32)
          + jnp.dot(add_ref[...], w1b_ref[...], preferred_element_type=jnp.float32)
          + b1_ref[...])
    h1 = jnp.where(h1 > 0, h1, 0.01 * h1)
    h2 = jnp.dot(h1, w2_ref[...], preferred_element_type=jnp.float32) + b2_ref[...]
    h2 = jnp.where(h2 > 0, h2, 0.01 * h2)
    out_ref[...] = jnp.dot(h2, w3_ref[...],
                           preferred_element_type=jnp.float32) + b3_ref[...]


_f32 = jnp.float32

_round0_call = pl.pallas_call(
    _round0_body,
    out_shape=[jax.ShapeDtypeStruct((_N, _H), _f32),
               jax.ShapeDtypeStruct((_N, 1), _f32),
               jax.ShapeDtypeStruct((_N, 1), _f32)],
)
_round_relu_call = pl.pallas_call(
    functools.partial(_round_body, True),
    out_shape=jax.ShapeDtypeStruct((_N, _H), _f32),
)
_round_norelu_call = pl.pallas_call(
    functools.partial(_round_body, False),
    out_shape=jax.ShapeDtypeStruct((_N, _H), _f32),
)
_epilogue_call = pl.pallas_call(
    _epilogue_body,
    out_shape=jax.ShapeDtypeStruct((_B, 1), _f32),
)


def kernel(x, edge_index, graph_ids, add_features, W0, b0, gcr_W, gcr_b,
           rW1, rb1, rW2, rb2, rW3, rb3):
    src = edge_index[0]
    dst = edge_index[1]
    _deg_kernel, _msg_kernel = _sc_kernels()

    ones_rows = jnp.ones((_CH, _H), _f32)
    zeros = jnp.zeros((_RPT, _H), _f32)
    src3 = src.reshape(_NW, _NCH, _CH)
    dst3 = dst.reshape(_NW, _NCH, _CH)
    degp = _deg_kernel(src3, dst3, ones_rows, zeros)
    dgo = degp[:, 0, :_N, 0]
    dgi = degp[:, 1, :_N, 0]

    h, dO, dI = _round0_call(x, W0, dgo, dgi)

    parts = _msg_kernel(h, src, dst3, zeros)
    for k in range(1, 11):
        i, j = divmod(k - 1, 2)
        W = gcr_W[i, j]
        b_prev = b0.reshape(1, _H) if k == 1 else gcr_b[(k - 2) // 2, (k - 2) % 2].reshape(1, _H)
        call = _round_norelu_call if k == 1 else _round_relu_call
        h = call(parts, dI, dO, b_prev, W)
        parts = _msg_kernel(h, src, dst3, zeros)

    b_last = gcr_b[4, 1].reshape(1, _H)
    gid_row = graph_ids.reshape(1, _N)
    out = _epilogue_call(parts, dI, b_last, gid_row, add_features,
                         rW1[:_H], rW1[_H:], rb1.reshape(1, 1024),
                         rW2, rb2.reshape(1, 512), rW3, rb3.reshape(1, 1))
    return out[:, 0]


# speedup vs baseline: 1.3031x; 1.0137x over previous
import functools

import jax
import jax.numpy as jnp
from jax import lax
from jax.experimental import pallas as pl
from jax.experimental.pallas import tpu as pltpu
from jax.experimental.pallas import tpu_sc as plsc

_N = 10000
_E = 320000
_H = 128
_B = 16
_NW = 32
_EPW = _E // _NW
_CH = 80
_NCH = _EPW // _CH
_NP = 10240
_RPT = _NP // 16

_SC_CACHE = {}


def _sc_kernels():
    if "deg" in _SC_CACHE:
        return _SC_CACHE["deg"], _SC_CACHE["msg"]

    mesh = plsc.VectorSubcoreMesh(core_axis_name="c", subcore_axis_name="s")

    @functools.partial(
        pl.kernel,
        mesh=mesh,
        out_type=jax.ShapeDtypeStruct((2, 2, _NP, _H), jnp.float32),
        scratch_types=[
            pltpu.VMEM((_NCH, _CH), jnp.int32),
            pltpu.VMEM((_NCH, _CH), jnp.int32),
            pltpu.VMEM((_CH, _H), jnp.float32),
            pltpu.VMEM_SHARED((_NP, _H), jnp.float32),
            pltpu.SemaphoreType.DMA,
        ],
    )
    def deg_kernel(src_hbm, dst_hbm, ones_hbm, zero_hbm, out_hbm,
                   srcm, dstm, ones_v, acc, sem):
        cid = lax.axis_index("c")
        sid = lax.axis_index("s")
        wid = cid * 16 + sid
        pltpu.sync_copy(ones_hbm, ones_v)
        pltpu.sync_copy(src_hbm.at[wid], srcm)
        pltpu.sync_copy(dst_hbm.at[wid], dstm)
        my_rows = pl.ds(sid * _RPT, _RPT)

        for which, em in ((0, srcm), (1, dstm)):
            pltpu.sync_copy(zero_hbm, acc.at[my_rows])
            plsc.subcore_barrier()

            def scat(g):
                pltpu.async_copy(ones_v, acc.at[em.at[g]], sem, add=True)

            def swait():
                pltpu.make_async_copy(ones_v, acc.at[pl.ds(0, _CH)],
                                      sem).wait()

            for g in range(4):
                scat(g)

            def body(g, carry):
                scat(g + 4)
                swait()
                return carry

            lax.fori_loop(0, _NCH - 4, body, 0)
            for _ in range(4):
                swait()
            plsc.subcore_barrier()
            pltpu.sync_copy(acc.at[my_rows],
                            out_hbm.at[cid, which, my_rows])

    @functools.partial(
        pl.kernel,
        mesh=mesh,
        out_type=jax.ShapeDtypeStruct((2, _NP, _H), jnp.float32),
        scratch_types=[
            pltpu.VMEM((_EPW,), jnp.int32),
            pltpu.VMEM((_NCH, _CH), jnp.int32),
            pltpu.VMEM((_CH, _H), jnp.float32),
            pltpu.VMEM((_CH, _H), jnp.float32),
            pltpu.VMEM_SHARED((_NP, _H), jnp.float32),
            pltpu.SemaphoreType.DMA,
            pltpu.SemaphoreType.DMA,
            pltpu.SemaphoreType.DMA,
        ],
    )
    def msg_kernel(h_hbm, src_hbm, dst_hbm, zero_hbm, out_hbm, srcv, dstm,
                   rows0, rows1, acc, sem0, sem1, zsem):
        cid = lax.axis_index("c")
        sid = lax.axis_index("s")
        wid = cid * 16 + sid
        pltpu.async_copy(zero_hbm, acc.at[pl.ds(sid * _RPT, _RPT)], zsem)
        pltpu.sync_copy(src_hbm.at[pl.ds(wid * _EPW, _EPW)], srcv)
        pltpu.sync_copy(dst_hbm.at[wid], dstm)

        def gather(g, rows, sem):
            return pltpu.async_copy(h_hbm.at[srcv.at[pl.ds(g * _CH, _CH)]],
                                    rows, sem)

        def gwait(rows, sem):
            pltpu.make_async_copy(h_hbm.at[pl.ds(0, _CH)], rows, sem).wait()

        def scat(g, rows):
            pltpu.sync_copy(rows, acc.at[dstm.at[g]], add=True)

        gather(0, rows0, sem0)
        pltpu.make_async_copy(zero_hbm, acc.at[pl.ds(sid * _RPT, _RPT)],
                              zsem).wait()
        plsc.subcore_barrier()

        def pair(t, carry):
            g0 = 2 * t
            gather(g0 + 1, rows1, sem1)
            gwait(rows0, sem0)
            scat(g0, rows0)
            gather(g0 + 2, rows0, sem0)
            gwait(rows1, sem1)
            scat(g0 + 1, rows1)
            return carry

        lax.fori_loop(0, (_NCH - 1) // 2, pair, 0)
        gwait(rows0, sem0)
        scat(_NCH - 1, rows0)

        plsc.subcore_barrier()
        pltpu.sync_copy(acc.at[pl.ds(sid * _RPT, _RPT)],
                        out_hbm.at[cid, pl.ds(sid * _RPT, _RPT)])

    _SC_CACHE["deg"] = deg_kernel
    _SC_CACHE["msg"] = msg_kernel
    return deg_kernel, msg_kernel


def _round0_body(x_ref, W_ref, dgo_ref, dgi_ref, h_ref, dO_ref, dI_ref):
    d_out = jnp.sum(dgo_ref[...], axis=0)
    d_in = jnp.sum(dgi_ref[...], axis=0)
    dO = lax.rsqrt(jnp.maximum(d_out, 1.0)).reshape(_N, 1)
    dI = lax.rsqrt(jnp.maximum(d_in, 1.0)).reshape(_N, 1)
    dO_ref[...] = dO
    dI_ref[...] = dI
    h_ref[...] = jnp.dot(x_ref[...] * dO, W_ref[...],
                         preferred_element_type=jnp.float32)


def _round_body(relu, parts_ref, dI_ref, dO_ref, b_ref, W_ref, h_ref):
    f = parts_ref[0, :_N, :] + parts_ref[1, :_N, :]
    f = f * dI_ref[...] + b_ref[...]
    if relu:
        f = jnp.maximum(f, 0.0)
    h_ref[...] = jnp.dot(f * dO_ref[...], W_ref[...],
                         preferred_element_type=jnp.float32)


def _epilogue_body(parts_ref, dI_ref, b_ref, gid_ref, add_ref,
                   w1a_ref, w1b_ref, b1_ref, w2_ref, b2_ref, w3_ref, b3_ref,
                   out_ref):
    f = parts_ref[0, :_N, :] + parts_ref[1, :_N, :]
    f = jnp.maximum(f * dI_ref[...] + b_ref[...], 0.0)
    iota = lax.broadcasted_iota(jnp.int32, (_B, 1), 0)
    onehot = (gid_ref[...] == iota).astype(jnp.float32)
    sums = jnp.dot(onehot, f, preferred_element_type=jnp.float32)
    counts = jnp.maximum(jnp.sum(onehot, axis=1), 1.0).reshape(_B, 1)
    mean = sums / counts
    h1 = (jnp.dot(mean, w1a_ref[...], preferred_element_type=jnp.float32)
          + jnp.dot(add_ref[...], w1b_ref[...], preferred_element_type=jnp.float32)
          + b1_ref[...])
    h1 = jnp.where(h1 > 0, h1, 0.01 * h1)
    h2 = jnp.dot(h1, w2_ref[...], preferred_element_type=jnp.float32) + b2_ref[...]
    h2 = jnp.where(h2 > 0, h2, 0.01 * h2)
    out_ref[...] = jnp.dot(h2, w3_ref[...],
                           preferred_element_type=jnp.float32) + b3_ref[...]


_f32 = jnp.float32

_round0_call = pl.pallas_call(
    _round0_body,
    out_shape=[jax.ShapeDtypeStruct((_N, _H), _f32),
               jax.ShapeDtypeStruct((_N, 1), _f32),
               jax.ShapeDtypeStruct((_N, 1), _f32)],
)
_round_relu_call = pl.pallas_call(
    functools.partial(_round_body, True),
    out_shape=jax.ShapeDtypeStruct((_N, _H), _f32),
)
_round_norelu_call = pl.pallas_call(
    functools.partial(_round_body, False),
    out_shape=jax.ShapeDtypeStruct((_N, _H), _f32),
)
_epilogue_call = pl.pallas_call(
    _epilogue_body,
    out_shape=jax.ShapeDtypeStruct((_B, 1), _f32),
)


def kernel(x, edge_index, graph_ids, add_features, W0, b0, gcr_W, gcr_b,
           rW1, rb1, rW2, rb2, rW3, rb3):
    src = edge_index[0]
    dst = edge_index[1]
    _deg_kernel, _msg_kernel = _sc_kernels()

    ones_rows = jnp.ones((_CH, _H), _f32)
    zeros = jnp.zeros((_RPT, _H), _f32)
    src3 = src.reshape(_NW, _NCH, _CH)
    dst3 = dst.reshape(_NW, _NCH, _CH)
    degp = _deg_kernel(src3, dst3, ones_rows, zeros)
    dgo = degp[:, 0, :_N, 0]
    dgi = degp[:, 1, :_N, 0]

    h, dO, dI = _round0_call(x, W0, dgo, dgi)

    parts = _msg_kernel(h, src, dst3, zeros)
    for k in range(1, 11):
        i, j = divmod(k - 1, 2)
        W = gcr_W[i, j]
        b_prev = b0.reshape(1, _H) if k == 1 else gcr_b[(k - 2) // 2, (k - 2) % 2].reshape(1, _H)
        call = _round_norelu_call if k == 1 else _round_relu_call
        h = call(parts, dI, dO, b_prev, W)
        parts = _msg_kernel(h, src, dst3, zeros)

    b_last = gcr_b[4, 1].reshape(1, _H)
    gid_row = graph_ids.reshape(1, _N)
    out = _epilogue_call(parts, dI, b_last, gid_row, add_features,
                         rW1[:_H], rW1[_H:], rb1.reshape(1, 1024),
                         rW2, rb2.reshape(1, 512), rW3, rb3.reshape(1, 1))
    return out[:, 0]


# speedup vs baseline: 1.3857x; 1.0634x over previous
import functools

import jax
import jax.numpy as jnp
from jax import lax
from jax.experimental import pallas as pl
from jax.experimental.pallas import tpu as pltpu
from jax.experimental.pallas import tpu_sc as plsc

_N = 10000
_E = 320000
_H = 128
_B = 16
_NW = 32
_EPW = _E // _NW
_CH = 80
_NCH = _EPW // _CH
_CHM = 104
_NM = 96
_TL = _EPW - _NM * _CHM
_NP = 10240
_RPT = _NP // 16

_SC_CACHE = {}


def _sc_kernels():
    if "deg" in _SC_CACHE:
        return _SC_CACHE["deg"], _SC_CACHE["msg"]

    mesh = plsc.VectorSubcoreMesh(core_axis_name="c", subcore_axis_name="s")

    @functools.partial(
        pl.kernel,
        mesh=mesh,
        out_type=jax.ShapeDtypeStruct((2, 2, _NP, _H), jnp.float32),
        scratch_types=[
            pltpu.VMEM((_NCH, _CH), jnp.int32),
            pltpu.VMEM((_NCH, _CH), jnp.int32),
            pltpu.VMEM((_CH, _H), jnp.float32),
            pltpu.VMEM_SHARED((_NP, _H), jnp.float32),
            pltpu.SemaphoreType.DMA,
        ],
    )
    def deg_kernel(src_hbm, dst_hbm, ones_hbm, zero_hbm, out_hbm,
                   srcm, dstm, ones_v, acc, sem):
        cid = lax.axis_index("c")
        sid = lax.axis_index("s")
        wid = cid * 16 + sid
        pltpu.sync_copy(ones_hbm, ones_v)
        pltpu.sync_copy(src_hbm.at[wid], srcm)
        pltpu.sync_copy(dst_hbm.at[wid], dstm)
        my_rows = pl.ds(sid * _RPT, _RPT)

        for which, em in ((0, srcm), (1, dstm)):
            pltpu.sync_copy(zero_hbm, acc.at[my_rows])
            plsc.subcore_barrier()

            def scat(g):
                pltpu.async_copy(ones_v, acc.at[em.at[g]], sem, add=True)

            def swait():
                pltpu.make_async_copy(ones_v, acc.at[pl.ds(0, _CH)],
                                      sem).wait()

            for g in range(4):
                scat(g)

            def body(g, carry):
                scat(g + 4)
                swait()
                return carry

            lax.fori_loop(0, _NCH - 4, body, 0)
            for _ in range(4):
                swait()
            plsc.subcore_barrier()
            pltpu.sync_copy(acc.at[my_rows],
                            out_hbm.at[cid, which, my_rows])

    @functools.partial(
        pl.kernel,
        mesh=mesh,
        out_type=jax.ShapeDtypeStruct((2, _NP, _H), jnp.float32),
        scratch_types=[
            pltpu.VMEM((_EPW,), jnp.int32),
            pltpu.VMEM((_NM, _CHM), jnp.int32),
            pltpu.VMEM((_TL,), jnp.int32),
            pltpu.VMEM((_CHM, _H), jnp.float32),
            pltpu.VMEM((_CHM, _H), jnp.float32),
            pltpu.VMEM_SHARED((_NP, _H), jnp.float32),
            pltpu.SemaphoreType.DMA,
            pltpu.SemaphoreType.DMA,
            pltpu.SemaphoreType.DMA,
        ],
    )
    def msg_kernel(h_hbm, src_hbm, dst_hbm, dstt_hbm, zero_hbm, out_hbm,
                   srcv, dstm, dstt, rows0, rows1, acc, sem0, sem1, zsem):
        cid = lax.axis_index("c")
        sid = lax.axis_index("s")
        wid = cid * 16 + sid
        pltpu.async_copy(zero_hbm, acc.at[pl.ds(sid * _RPT, _RPT)], zsem)
        pltpu.sync_copy(src_hbm.at[pl.ds(wid * _EPW, _EPW)], srcv)
        pltpu.sync_copy(dst_hbm.at[wid], dstm)
        pltpu.sync_copy(dstt_hbm.at[wid], dstt)

        def gather(g, rows, sem):
            return pltpu.async_copy(h_hbm.at[srcv.at[pl.ds(g * _CHM, _CHM)]],
                                    rows, sem)

        def gwait(rows, sem):
            pltpu.make_async_copy(h_hbm.at[pl.ds(0, _CHM)], rows, sem).wait()

        def scat(g, rows):
            pltpu.sync_copy(rows, acc.at[dstm.at[g]], add=True)

        gather(0, rows0, sem0)
        pltpu.make_async_copy(zero_hbm, acc.at[pl.ds(sid * _RPT, _RPT)],
                              zsem).wait()
        plsc.subcore_barrier()

        def pair(t, carry):
            g0 = 2 * t
            gather(g0 + 1, rows1, sem1)
            gwait(rows0, sem0)
            scat(g0, rows0)
            gather(g0 + 2, rows0, sem0)
            gwait(rows1, sem1)
            scat(g0 + 1, rows1)
            return carry

        lax.fori_loop(0, (_NM - 2) // 2, pair, 0)
        gather(_NM - 1, rows1, sem1)
        gwait(rows0, sem0)
        scat(_NM - 2, rows0)
        tail = rows0.at[pl.ds(0, _TL)]
        pltpu.async_copy(
            h_hbm.at[srcv.at[pl.ds(_NM * _CHM, _TL)]], tail, sem0)
        gwait(rows1, sem1)
        scat(_NM - 1, rows1)
        pltpu.make_async_copy(h_hbm.at[pl.ds(0, _TL)], tail, sem0).wait()
        pltpu.sync_copy(tail, acc.at[dstt], add=True)

        plsc.subcore_barrier()
        pltpu.sync_copy(acc.at[pl.ds(sid * _RPT, _RPT)],
                        out_hbm.at[cid, pl.ds(sid * _RPT, _RPT)])

    _SC_CACHE["deg"] = deg_kernel
    _SC_CACHE["msg"] = msg_kernel
    return deg_kernel, msg_kernel


def _round0_body(x_ref, W_ref, dgo_ref, dgi_ref, h_ref, dO_ref, dI_ref):
    d_out = jnp.sum(dgo_ref[...], axis=0)
    d_in = jnp.sum(dgi_ref[...], axis=0)
    dO = lax.rsqrt(jnp.maximum(d_out, 1.0)).reshape(_N, 1)
    dI = lax.rsqrt(jnp.maximum(d_in, 1.0)).reshape(_N, 1)
    dO_ref[...] = dO
    dI_ref[...] = dI
    h_ref[...] = jnp.dot(x_ref[...] * dO, W_ref[...],
                         preferred_element_type=jnp.float32)


def _round_body(relu, parts_ref, dI_ref, dO_ref, b_ref, W_ref, h_ref):
    f = parts_ref[0, :_N, :] + parts_ref[1, :_N, :]
    f = f * dI_ref[...] + b_ref[...]
    if relu:
        f = jnp.maximum(f, 0.0)
    h_ref[...] = jnp.dot(f * dO_ref[...], W_ref[...],
                         preferred_element_type=jnp.float32)


def _epilogue_body(parts_ref, dI_ref, b_ref, gid_ref, add_ref,
                   w1a_ref, w1b_ref, b1_ref, w2_ref, b2_ref, w3_ref, b3_ref,
                   out_ref):
    f = parts_ref[0, :_N, :] + parts_ref[1, :_N, :]
    f = jnp.maximum(f * dI_ref[...] + b_ref[...], 0.0)
    iota = lax.broadcasted_iota(jnp.int32, (_B, 1), 0)
    onehot = (gid_ref[...] == iota).astype(jnp.float32)
    sums = jnp.dot(onehot, f, preferred_element_type=jnp.float32)
    counts = jnp.maximum(jnp.sum(onehot, axis=1), 1.0).reshape(_B, 1)
    mean = sums / counts
    h1 = (jnp.dot(mean, w1a_ref[...], preferred_element_type=jnp.float32)
          + jnp.dot(add_ref[...], w1b_ref[...], preferred_element_type=jnp.float32)
          + b1_ref[...])
    h1 = jnp.where(h1 > 0, h1, 0.01 * h1)
    h2 = jnp.dot(h1, w2_ref[...], preferred_element_type=jnp.float32) + b2_ref[...]
    h2 = jnp.where(h2 > 0, h2, 0.01 * h2)
    out_ref[...] = jnp.dot(h2, w3_ref[...],
                           preferred_element_type=jnp.float32) + b3_ref[...]


_f32 = jnp.float32

_round0_call = pl.pallas_call(
    _round0_body,
    out_shape=[jax.ShapeDtypeStruct((_N, _H), _f32),
               jax.ShapeDtypeStruct((_N, 1), _f32),
               jax.ShapeDtypeStruct((_N, 1), _f32)],
)
_round_relu_call = pl.pallas_call(
    functools.partial(_round_body, True),
    out_shape=jax.ShapeDtypeStruct((_N, _H), _f32),
)
_round_norelu_call = pl.pallas_call(
    functools.partial(_round_body, False),
    out_shape=jax.ShapeDtypeStruct((_N, _H), _f32),
)
_epilogue_call = pl.pallas_call(
    _epilogue_body,
    out_shape=jax.ShapeDtypeStruct((_B, 1), _f32),
)


def kernel(x, edge_index, graph_ids, add_features, W0, b0, gcr_W, gcr_b,
           rW1, rb1, rW2, rb2, rW3, rb3):
    src = edge_index[0]
    dst = edge_index[1]
    _deg_kernel, _msg_kernel = _sc_kernels()

    ones_rows = jnp.ones((_CH, _H), _f32)
    zeros = jnp.zeros((_RPT, _H), _f32)
    src3 = src.reshape(_NW, _NCH, _CH)
    dst3 = dst.reshape(_NW, _NCH, _CH)
    degp = _deg_kernel(src3, dst3, ones_rows, zeros)
    dgo = degp[:, 0, :_N, 0]
    dgi = degp[:, 1, :_N, 0]

    h, dO, dI = _round0_call(x, W0, dgo, dgi)
    dst2 = dst.reshape(_NW, _EPW)
    dstm3 = dst2[:, :_NM * _CHM].reshape(_NW, _NM, _CHM)
    dstt2 = dst2[:, _NM * _CHM:]

    parts = _msg_kernel(h, src, dstm3, dstt2, zeros)
    for k in range(1, 11):
        i, j = divmod(k - 1, 2)
        W = gcr_W[i, j]
        b_prev = b0.reshape(1, _H) if k == 1 else gcr_b[(k - 2) // 2, (k - 2) % 2].reshape(1, _H)
        call = _round_norelu_call if k == 1 else _round_relu_call
        h = call(parts, dI, dO, b_prev, W)
        parts = _msg_kernel(h, src, dstm3, dstt2, zeros)

    b_last = gcr_b[4, 1].reshape(1, _H)
    gid_row = graph_ids.reshape(1, _N)
    out = _epilogue_call(parts, dI, b_last, gid_row, add_features,
                         rW1[:_H], rW1[_H:], rb1.reshape(1, 1024),
                         rW2, rb2.reshape(1, 512), rW3, rb3.reshape(1, 1))
    return out[:, 0]


# speedup vs baseline: 1.3896x; 1.0028x over previous
import functools

import jax
import jax.numpy as jnp
from jax import lax
from jax.experimental import pallas as pl
from jax.experimental.pallas import tpu as pltpu
from jax.experimental.pallas import tpu_sc as plsc

_N = 10000
_E = 320000
_H = 128
_B = 16
_NW = 32
_EPW = _E // _NW
_CH = 80
_NCH = _EPW // _CH
_CHM = 96
_NM = 104
_TL = _EPW - _NM * _CHM
_NP = 10240
_RPT = _NP // 16

_SC_CACHE = {}


def _sc_kernels():
    if "deg" in _SC_CACHE:
        return _SC_CACHE["deg"], _SC_CACHE["msg"]

    mesh = plsc.VectorSubcoreMesh(core_axis_name="c", subcore_axis_name="s")

    @functools.partial(
        pl.kernel,
        mesh=mesh,
        out_type=jax.ShapeDtypeStruct((2, 2, _NP, _H), jnp.float32),
        scratch_types=[
            pltpu.VMEM((_NCH, _CH), jnp.int32),
            pltpu.VMEM((_NCH, _CH), jnp.int32),
            pltpu.VMEM((_CH, _H), jnp.float32),
            pltpu.VMEM_SHARED((_NP, _H), jnp.float32),
            pltpu.SemaphoreType.DMA,
        ],
    )
    def deg_kernel(src_hbm, dst_hbm, ones_hbm, zero_hbm, out_hbm,
                   srcm, dstm, ones_v, acc, sem):
        cid = lax.axis_index("c")
        sid = lax.axis_index("s")
        wid = cid * 16 + sid
        pltpu.sync_copy(ones_hbm, ones_v)
        pltpu.sync_copy(src_hbm.at[wid], srcm)
        pltpu.sync_copy(dst_hbm.at[wid], dstm)
        my_rows = pl.ds(sid * _RPT, _RPT)

        for which, em in ((0, srcm), (1, dstm)):
            pltpu.sync_copy(zero_hbm, acc.at[my_rows])
            plsc.subcore_barrier()

            def scat(g):
                pltpu.async_copy(ones_v, acc.at[em.at[g]], sem, add=True)

            def swait():
                pltpu.make_async_copy(ones_v, acc.at[pl.ds(0, _CH)],
                                      sem).wait()

            for g in range(4):
                scat(g)

            def body(g, carry):
                scat(g + 4)
                swait()
                return carry

            lax.fori_loop(0, _NCH - 4, body, 0)
            for _ in range(4):
                swait()
            plsc.subcore_barrier()
            pltpu.sync_copy(acc.at[my_rows],
                            out_hbm.at[cid, which, my_rows])

    @functools.partial(
        pl.kernel,
        mesh=mesh,
        out_type=jax.ShapeDtypeStruct((2, _NP, _H), jnp.float32),
        scratch_types=[
            pltpu.VMEM((_EPW,), jnp.int32),
            pltpu.VMEM((_EPW,), jnp.int32),
            pltpu.VMEM((_CHM, _H), jnp.float32),
            pltpu.VMEM((_CHM, _H), jnp.float32),
            pltpu.VMEM_SHARED((_NP, _H), jnp.float32),
            pltpu.SemaphoreType.DMA,
            pltpu.SemaphoreType.DMA,
            pltpu.SemaphoreType.DMA,
            pltpu.SemaphoreType.DMA,
        ],
    )
    def msg_kernel(h_hbm, src_hbm, dst_hbm, zero_hbm, out_hbm,
                   srcv, dstv, rows0, rows1, acc, sem0, sem1, zsem, ssem):
        cid = lax.axis_index("c")
        sid = lax.axis_index("s")
        wid = cid * 16 + sid
        pltpu.async_copy(zero_hbm, acc.at[pl.ds(sid * _RPT, _RPT)], zsem)
        pltpu.sync_copy(src_hbm.at[pl.ds(wid * _EPW, _EPW)], srcv)
        pltpu.sync_copy(dst_hbm.at[pl.ds(wid * _EPW, _EPW)], dstv)

        def gather(g, rows, sem):
            return pltpu.async_copy(h_hbm.at[srcv.at[pl.ds(g * _CHM, _CHM)]],
                                    rows, sem)

        def gwait(rows, sem):
            pltpu.make_async_copy(h_hbm.at[pl.ds(0, _CHM)], rows, sem).wait()

        def scat(g, rows):
            for k in range(_CHM // 16):
                idx = dstv[pl.ds(g * _CHM + k * 16, 16)]
                pltpu.async_copy(rows.at[pl.ds(k * 16, 16)], acc.at[idx],
                                 ssem, add=True)
            for k in range(_CHM // 16):
                pltpu.make_async_copy(rows.at[pl.ds(0, 16)],
                                      acc.at[pl.ds(0, 16)], ssem).wait()

        gather(0, rows0, sem0)
        pltpu.make_async_copy(zero_hbm, acc.at[pl.ds(sid * _RPT, _RPT)],
                              zsem).wait()
        plsc.subcore_barrier()

        def pair(t, carry):
            g0 = 2 * t
            gather(g0 + 1, rows1, sem1)
            gwait(rows0, sem0)
            scat(g0, rows0)
            gather(g0 + 2, rows0, sem0)
            gwait(rows1, sem1)
            scat(g0 + 1, rows1)
            return carry

        lax.fori_loop(0, (_NM - 2) // 2, pair, 0)
        gather(_NM - 1, rows1, sem1)
        gwait(rows0, sem0)
        scat(_NM - 2, rows0)
        tail = rows0.at[pl.ds(0, _TL)]
        pltpu.async_copy(
            h_hbm.at[srcv.at[pl.ds(_NM * _CHM, _TL)]], tail, sem0)
        gwait(rows1, sem1)
        scat(_NM - 1, rows1)
        pltpu.make_async_copy(h_hbm.at[pl.ds(0, _TL)], tail, sem0).wait()
        tidx = dstv[pl.ds(_NM * _CHM, _TL)]
        pltpu.async_copy(tail, acc.at[tidx], ssem, add=True)
        pltpu.make_async_copy(rows0.at[pl.ds(0, 16)],
                              acc.at[pl.ds(0, 16)], ssem).wait()

        plsc.subcore_barrier()
        pltpu.sync_copy(acc.at[pl.ds(sid * _RPT, _RPT)],
                        out_hbm.at[cid, pl.ds(sid * _RPT, _RPT)])

    _SC_CACHE["deg"] = deg_kernel
    _SC_CACHE["msg"] = msg_kernel
    return deg_kernel, msg_kernel


def _round0_body(x_ref, W_ref, dgo_ref, dgi_ref, h_ref, dO_ref, dI_ref):
    d_out = jnp.sum(dgo_ref[...], axis=0)
    d_in = jnp.sum(dgi_ref[...], axis=0)
    dO = lax.rsqrt(jnp.maximum(d_out, 1.0)).reshape(_N, 1)
    dI = lax.rsqrt(jnp.maximum(d_in, 1.0)).reshape(_N, 1)
    dO_ref[...] = dO
    dI_ref[...] = dI
    h_ref[...] = jnp.dot(x_ref[...] * dO, W_ref[...],
                         preferred_element_type=jnp.float32)


def _round_body(relu, parts_ref, dI_ref, dO_ref, b_ref, W_ref, h_ref):
    f = parts_ref[0, :_N, :] + parts_ref[1, :_N, :]
    f = f * dI_ref[...] + b_ref[...]
    if relu:
        f = jnp.maximum(f, 0.0)
    h_ref[...] = jnp.dot(f * dO_ref[...], W_ref[...],
                         preferred_element_type=jnp.float32)


def _epilogue_body(parts_ref, dI_ref, b_ref, gid_ref, add_ref,
                   w1a_ref, w1b_ref, b1_ref, w2_ref, b2_ref, w3_ref, b3_ref,
                   out_ref):
    f = parts_ref[0, :_N, :] + parts_ref[1, :_N, :]
    f = jnp.maximum(f * dI_ref[...] + b_ref[...], 0.0)
    iota = lax.broadcasted_iota(jnp.int32, (_B, 1), 0)
    onehot = (gid_ref[...] == iota).astype(jnp.float32)
    sums = jnp.dot(onehot, f, preferred_element_type=jnp.float32)
    counts = jnp.maximum(jnp.sum(onehot, axis=1), 1.0).reshape(_B, 1)
    mean = sums / counts
    h1 = (jnp.dot(mean, w1a_ref[...], preferred_element_type=jnp.float32)
          + jnp.dot(add_ref[...], w1b_ref[...], preferred_element_type=jnp.float32)
          + b1_ref[...])
    h1 = jnp.where(h1 > 0, h1, 0.01 * h1)
    h2 = jnp.dot(h1, w2_ref[...], preferred_element_type=jnp.float32) + b2_ref[...]
    h2 = jnp.where(h2 > 0, h2, 0.01 * h2)
    out_ref[...] = jnp.dot(h2, w3_ref[...],
                           preferred_element_type=jnp.float32) + b3_ref[...]


_f32 = jnp.float32

_round0_call = pl.pallas_call(
    _round0_body,
    out_shape=[jax.ShapeDtypeStruct((_N, _H), _f32),
               jax.ShapeDtypeStruct((_N, 1), _f32),
               jax.ShapeDtypeStruct((_N, 1), _f32)],
)
_round_relu_call = pl.pallas_call(
    functools.partial(_round_body, True),
    out_shape=jax.ShapeDtypeStruct((_N, _H), _f32),
)
_round_norelu_call = pl.pallas_call(
    functools.partial(_round_body, False),
    out_shape=jax.ShapeDtypeStruct((_N, _H), _f32),
)
_epilogue_call = pl.pallas_call(
    _epilogue_body,
    out_shape=jax.ShapeDtypeStruct((_B, 1), _f32),
)


def kernel(x, edge_index, graph_ids, add_features, W0, b0, gcr_W, gcr_b,
           rW1, rb1, rW2, rb2, rW3, rb3):
    src = edge_index[0]
    dst = edge_index[1]
    _deg_kernel, _msg_kernel = _sc_kernels()

    ones_rows = jnp.ones((_CH, _H), _f32)
    zeros = jnp.zeros((_RPT, _H), _f32)
    src3 = src.reshape(_NW, _NCH, _CH)
    dst3 = dst.reshape(_NW, _NCH, _CH)
    degp = _deg_kernel(src3, dst3, ones_rows, zeros)
    dgo = degp[:, 0, :_N, 0]
    dgi = degp[:, 1, :_N, 0]

    h, dO, dI = _round0_call(x, W0, dgo, dgi)

    parts = _msg_kernel(h, src, dst, zeros)
    for k in range(1, 11):
        i, j = divmod(k - 1, 2)
        W = gcr_W[i, j]
        b_prev = b0.reshape(1, _H) if k == 1 else gcr_b[(k - 2) // 2, (k - 2) % 2].reshape(1, _H)
        call = _round_norelu_call if k == 1 else _round_relu_call
        h = call(parts, dI, dO, b_prev, W)
        parts = _msg_kernel(h, src, dst, zeros)

    b_last = gcr_b[4, 1].reshape(1, _H)
    gid_row = graph_ids.reshape(1, _N)
    out = _epilogue_call(parts, dI, b_last, gid_row, add_features,
                         rW1[:_H], rW1[_H:], rb1.reshape(1, 1024),
                         rW2, rb2.reshape(1, 512), rW3, rb3.reshape(1, 1))
    return out[:, 0]


# speedup vs baseline: 1.4252x; 1.0256x over previous
import functools

import jax
import jax.numpy as jnp
from jax import lax
from jax.experimental import pallas as pl
from jax.experimental.pallas import tpu as pltpu
from jax.experimental.pallas import tpu_sc as plsc

_N = 10000
_E = 320000
_H = 128
_B = 16
_NW = 32
_EPW = _E // _NW
_CH = 80
_NCH = _EPW // _CH
_CHM = 64
_NM = 156
_TL = _EPW - _NM * _CHM
_SPC = _CHM // 16
_NP = 10240
_RPT = _NP // 16

_SC_CACHE = {}


def _sc_kernels():
    if "deg" in _SC_CACHE:
        return _SC_CACHE["deg"], _SC_CACHE["msg"]

    mesh = plsc.VectorSubcoreMesh(core_axis_name="c", subcore_axis_name="s")

    @functools.partial(
        pl.kernel,
        mesh=mesh,
        out_type=jax.ShapeDtypeStruct((2, 2, _NP, _H), jnp.float32),
        scratch_types=[
            pltpu.VMEM((_NCH, _CH), jnp.int32),
            pltpu.VMEM((_NCH, _CH), jnp.int32),
            pltpu.VMEM((_CH, _H), jnp.float32),
            pltpu.VMEM_SHARED((_NP, _H), jnp.float32),
            pltpu.SemaphoreType.DMA,
        ],
    )
    def deg_kernel(src_hbm, dst_hbm, ones_hbm, zero_hbm, out_hbm,
                   srcm, dstm, ones_v, acc, sem):
        cid = lax.axis_index("c")
        sid = lax.axis_index("s")
        wid = cid * 16 + sid
        pltpu.sync_copy(ones_hbm, ones_v)
        pltpu.sync_copy(src_hbm.at[wid], srcm)
        pltpu.sync_copy(dst_hbm.at[wid], dstm)
        my_rows = pl.ds(sid * _RPT, _RPT)

        for which, em in ((0, srcm), (1, dstm)):
            pltpu.sync_copy(zero_hbm, acc.at[my_rows])
            plsc.subcore_barrier()

            def scat(g):
                pltpu.async_copy(ones_v, acc.at[em.at[g]], sem, add=True)

            def swait():
                pltpu.make_async_copy(ones_v, acc.at[pl.ds(0, _CH)],
                                      sem).wait()

            for g in range(4):
                scat(g)

            def body(g, carry):
                scat(g + 4)
                swait()
                return carry

            lax.fori_loop(0, _NCH - 4, body, 0)
            for _ in range(4):
                swait()
            plsc.subcore_barrier()
            pltpu.sync_copy(acc.at[my_rows],
                            out_hbm.at[cid, which, my_rows])

    @functools.partial(
        pl.kernel,
        mesh=mesh,
        out_type=jax.ShapeDtypeStruct((2, _NP, _H), jnp.float32),
        scratch_types=[
            pltpu.VMEM((_EPW,), jnp.int32),
            pltpu.VMEM((_EPW,), jnp.int32),
            pltpu.VMEM((_CHM, _H), jnp.float32),
            pltpu.VMEM((_CHM, _H), jnp.float32),
            pltpu.VMEM((_CHM, _H), jnp.float32),
            pltpu.VMEM_SHARED((_NP, _H), jnp.float32),
            pltpu.SemaphoreType.DMA,
            pltpu.SemaphoreType.DMA,
            pltpu.SemaphoreType.DMA,
            pltpu.SemaphoreType.DMA,
            pltpu.SemaphoreType.DMA,
            pltpu.SemaphoreType.DMA,
            pltpu.SemaphoreType.DMA,
        ],
    )
    def msg_kernel(h_hbm, src_hbm, dst_hbm, zero_hbm, out_hbm,
                   srcv, dstv, rows0, rows1, rows2, acc,
                   sem0, sem1, sem2, ssem0, ssem1, ssem2, zsem):
        cid = lax.axis_index("c")
        sid = lax.axis_index("s")
        wid = cid * 16 + sid
        pltpu.async_copy(zero_hbm, acc.at[pl.ds(sid * _RPT, _RPT)], zsem)
        pltpu.sync_copy(src_hbm.at[pl.ds(wid * _EPW, _EPW)], srcv)
        pltpu.sync_copy(dst_hbm.at[pl.ds(wid * _EPW, _EPW)], dstv)

        rows = (rows0, rows1, rows2)
        gsem = (sem0, sem1, sem2)
        ssem = (ssem0, ssem1, ssem2)

        def gather(g, b):
            pltpu.async_copy(h_hbm.at[srcv.at[pl.ds(g * _CHM, _CHM)]],
                             rows[b], gsem[b])

        def gwait(b):
            pltpu.make_async_copy(h_hbm.at[pl.ds(0, _CHM)], rows[b],
                                  gsem[b]).wait()

        def fire(g, b):
            for k in range(_SPC):
                idx = dstv[pl.ds(g * _CHM + k * 16, 16)]
                pltpu.async_copy(rows[b].at[pl.ds(k * 16, 16)], acc.at[idx],
                                 ssem[b], add=True)

        def drain(b, n=_SPC):
            for _ in range(n):
                pltpu.make_async_copy(rows[b].at[pl.ds(0, 16)],
                                      acc.at[pl.ds(0, 16)], ssem[b]).wait()

        gather(0, 0)
        gather(1, 1)
        pltpu.make_async_copy(zero_hbm, acc.at[pl.ds(sid * _RPT, _RPT)],
                              zsem).wait()
        plsc.subcore_barrier()

        gwait(0)
        fire(0, 0)
        gather(2, 2)

        def trip(t, carry):
            g = 3 * t
            for j in (1, 2, 0):
                gwait(j)
                fire(g + j, j)
                nxt = g + j + 2
                b = (j + 2) % 3
                drain(b)
                gather(nxt, b)
            return carry

        lax.fori_loop(0, (_NM - 6) // 3, trip, 0)
        g0 = _NM - 5
        for i, j in enumerate((1, 2, 0, 1, 2)):
            g = g0 + i
            gwait(j)
            fire(g, j)
            nxt = g + 2
            b = (j + 2) % 3
            drain(b)
            if nxt < _NM:
                gather(nxt, b)
            elif nxt == _NM:
                pltpu.async_copy(
                    h_hbm.at[srcv.at[pl.ds(_NM * _CHM, _TL)]],
                    rows[b].at[pl.ds(0, _TL)], gsem[b])
        tb = 0
        pltpu.make_async_copy(h_hbm.at[pl.ds(0, _TL)],
                              rows[tb].at[pl.ds(0, _TL)], gsem[tb]).wait()
        tidx = dstv[pl.ds(_NM * _CHM, _TL)]
        pltpu.async_copy(rows[tb].at[pl.ds(0, _TL)], acc.at[tidx],
                         ssem[tb], add=True)
        drain(2)
        drain(tb, 1)

        plsc.subcore_barrier()
        pltpu.sync_copy(acc.at[pl.ds(sid * _RPT, _RPT)],
                        out_hbm.at[cid, pl.ds(sid * _RPT, _RPT)])

    _SC_CACHE["deg"] = deg_kernel
    _SC_CACHE["msg"] = msg_kernel
    return deg_kernel, msg_kernel


def _round0_body(x_ref, W_ref, dgo_ref, dgi_ref, h_ref, dO_ref, dI_ref):
    d_out = jnp.sum(dgo_ref[...], axis=0)
    d_in = jnp.sum(dgi_ref[...], axis=0)
    dO = lax.rsqrt(jnp.maximum(d_out, 1.0)).reshape(_N, 1)
    dI = lax.rsqrt(jnp.maximum(d_in, 1.0)).reshape(_N, 1)
    dO_ref[...] = dO
    dI_ref[...] = dI
    h_ref[...] = jnp.dot(x_ref[...] * dO, W_ref[...],
                         preferred_element_type=jnp.float32)


def _round_body(relu, parts_ref, dI_ref, dO_ref, b_ref, W_ref, h_ref):
    f = parts_ref[0, :_N, :] + parts_ref[1, :_N, :]
    f = f * dI_ref[...] + b_ref[...]
    if relu:
        f = jnp.maximum(f, 0.0)
    h_ref[...] = jnp.dot(f * dO_ref[...], W_ref[...],
                         preferred_element_type=jnp.float32)


def _epilogue_body(parts_ref, dI_ref, b_ref, gid_ref, add_ref,
                   w1a_ref, w1b_ref, b1_ref, w2_ref, b2_ref, w3_ref, b3_ref,
                   out_ref):
    f = parts_ref[0, :_N, :] + parts_ref[1, :_N, :]
    f = jnp.maximum(f * dI_ref[...] + b_ref[...], 0.0)
    iota = lax.broadcasted_iota(jnp.int32, (_B, 1), 0)
    onehot = (gid_ref[...] == iota).astype(jnp.float32)
    sums = jnp.dot(onehot, f, preferred_element_type=jnp.float32)
    counts = jnp.maximum(jnp.sum(onehot, axis=1), 1.0).reshape(_B, 1)
    mean = sums / counts
    h1 = (jnp.dot(mean, w1a_ref[...], preferred_element_type=jnp.float32)
          + jnp.dot(add_ref[...], w1b_ref[...], preferred_element_type=jnp.float32)
          + b1_ref[...])
    h1 = jnp.where(h1 > 0, h1, 0.01 * h1)
    h2 = jnp.dot(h1, w2_ref[...], preferred_element_type=jnp.float32) + b2_ref[...]
    h2 = jnp.where(h2 > 0, h2, 0.01 * h2)
    out_ref[...] = jnp.dot(h2, w3_ref[...],
                           preferred_element_type=jnp.float32) + b3_ref[...]


_f32 = jnp.float32

_round0_call = pl.pallas_call(
    _round0_body,
    out_shape=[jax.ShapeDtypeStruct((_N, _H), _f32),
               jax.ShapeDtypeStruct((_N, 1), _f32),
               jax.ShapeDtypeStruct((_N, 1), _f32)],
)
_round_relu_call = pl.pallas_call(
    functools.partial(_round_body, True),
    out_shape=jax.ShapeDtypeStruct((_N, _H), _f32),
)
_round_norelu_call = pl.pallas_call(
    functools.partial(_round_body, False),
    out_shape=jax.ShapeDtypeStruct((_N, _H), _f32),
)
_epilogue_call = pl.pallas_call(
    _epilogue_body,
    out_shape=jax.ShapeDtypeStruct((_B, 1), _f32),
)


def kernel(x, edge_index, graph_ids, add_features, W0, b0, gcr_W, gcr_b,
           rW1, rb1, rW2, rb2, rW3, rb3):
    src = edge_index[0]
    dst = edge_index[1]
    _deg_kernel, _msg_kernel = _sc_kernels()

    ones_rows = jnp.ones((_CH, _H), _f32)
    zeros = jnp.zeros((_RPT, _H), _f32)
    src3 = src.reshape(_NW, _NCH, _CH)
    dst3 = dst.reshape(_NW, _NCH, _CH)
    degp = _deg_kernel(src3, dst3, ones_rows, zeros)
    dgo = degp[:, 0, :_N, 0]
    dgi = degp[:, 1, :_N, 0]

    h, dO, dI = _round0_call(x, W0, dgo, dgi)

    parts = _msg_kernel(h, src, dst, zeros)
    for k in range(1, 11):
        i, j = divmod(k - 1, 2)
        W = gcr_W[i, j]
        b_prev = b0.reshape(1, _H) if k == 1 else gcr_b[(k - 2) // 2, (k - 2) % 2].reshape(1, _H)
        call = _round_norelu_call if k == 1 else _round_relu_call
        h = call(parts, dI, dO, b_prev, W)
        parts = _msg_kernel(h, src, dst, zeros)

    b_last = gcr_b[4, 1].reshape(1, _H)
    gid_row = graph_ids.reshape(1, _N)
    out = _epilogue_call(parts, dI, b_last, gid_row, add_features,
                         rW1[:_H], rW1[_H:], rb1.reshape(1, 1024),
                         rW2, rb2.reshape(1, 512), rW3, rb3.reshape(1, 1))
    return out[:, 0]
